# speedup vs baseline: 1.0923x; 1.0923x over previous
import functools

import jax
import jax.numpy as jnp
from jax import lax
from jax.experimental import pallas as pl
from jax.experimental.pallas import tpu as pltpu
from jax.experimental.pallas import tpu_sc as plsc

BATCH = 16384
LANES = 128
ROWS = BATCH // LANES
NC, NS = 2, 16
NW = NC * NS
RPW = ROWS // NW

_MESH = plsc.VectorSubcoreMesh(core_axis_name="c", subcore_axis_name="s")


@functools.partial(
    pl.kernel,
    out_type=jax.ShapeDtypeStruct((ROWS, LANES), jnp.float32),
    mesh=_MESH,
    scratch_types=[
        pltpu.VMEM((RPW, LANES), jnp.int32),
        pltpu.VMEM((RPW, LANES), jnp.float32),
        pltpu.SemaphoreType.DMA,
    ],
)
def _sc_gather(w_hbm, idx_hbm, out_hbm, idx_v, val_v, sem):
    wid = lax.axis_index("s") * NC + lax.axis_index("c")
    base = wid * RPW
    pltpu.sync_copy(idx_hbm.at[pl.ds(base, RPW)], idx_v)
    copies = [
        pltpu.async_copy(w_hbm.at[idx_v.at[j]], val_v.at[j], sem)
        for j in range(RPW)
    ]
    for c in copies:
        c.wait()
    pltpu.sync_copy(val_v, out_hbm.at[pl.ds(base, RPW)])


def kernel(weights, indices):
    idx = indices.astype(jnp.int32).reshape(ROWS, LANES)
    out = _sc_gather(weights, idx)
    return out.reshape(BATCH)


# speedup vs baseline: 1.1077x; 1.0141x over previous
import functools

import jax
import jax.numpy as jnp
from jax import lax
from jax.experimental import pallas as pl
from jax.experimental.pallas import tpu as pltpu
from jax.experimental.pallas import tpu_sc as plsc

BATCH = 16384
LANES = 128
ROWS = BATCH // LANES
NC, NS = 2, 16
NW = NC * NS
RPW = ROWS // NW

_MESH = plsc.VectorSubcoreMesh(core_axis_name="c", subcore_axis_name="s")


IPW = BATCH // NW


@functools.partial(
    pl.kernel,
    out_type=jax.ShapeDtypeStruct((BATCH,), jnp.float32),
    mesh=_MESH,
    scratch_types=[
        pltpu.VMEM((IPW,), jnp.int32),
        pltpu.VMEM((IPW,), jnp.float32),
        pltpu.SemaphoreType.DMA,
    ],
)
def _sc_gather(w_hbm, idx_hbm, out_hbm, idx_v, val_v, sem):
    wid = lax.axis_index("s") * NC + lax.axis_index("c")
    base = wid * IPW
    pltpu.sync_copy(idx_hbm.at[pl.ds(base, IPW)], idx_v)
    pltpu.async_copy(w_hbm.at[idx_v], val_v, sem).wait()
    pltpu.sync_copy(val_v, out_hbm.at[pl.ds(base, IPW)])


def kernel(weights, indices):
    return _sc_gather(weights, indices.astype(jnp.int32))


# speedup vs baseline: 1.1078x; 1.0001x over previous
import functools

import jax
import jax.numpy as jnp
from jax import lax
from jax.experimental import pallas as pl
from jax.experimental.pallas import tpu as pltpu
from jax.experimental.pallas import tpu_sc as plsc

BATCH = 16384
LANES = 128
ROWS = BATCH // LANES
NC, NS = 2, 16
NW = NC * NS
RPW = ROWS // NW

_MESH = plsc.VectorSubcoreMesh(core_axis_name="c", subcore_axis_name="s")


IPW = BATCH // NW


@functools.partial(
    pl.kernel,
    out_type=jax.ShapeDtypeStruct((BATCH,), jnp.float32),
    mesh=_MESH,
    scratch_types=[
        pltpu.VMEM((IPW,), jnp.int32),
        pltpu.VMEM((IPW,), jnp.float32),
        pltpu.SemaphoreType.DMA,
        pltpu.SemaphoreType.DMA,
        pltpu.SemaphoreType.DMA,
    ],
)
def _sc_gather(w_hbm, idx_hbm, out_hbm, idx_v, val_v, g0, g1, so):
    wid = lax.axis_index("s") * NC + lax.axis_index("c")
    base = wid * IPW
    half = IPW // 2
    pltpu.sync_copy(idx_hbm.at[pl.ds(base, IPW)], idx_v)
    c0 = pltpu.async_copy(w_hbm.at[idx_v.at[pl.ds(0, half)]],
                          val_v.at[pl.ds(0, half)], g0)
    c1 = pltpu.async_copy(w_hbm.at[idx_v.at[pl.ds(half, half)]],
                          val_v.at[pl.ds(half, half)], g1)
    c0.wait()
    s0 = pltpu.async_copy(val_v.at[pl.ds(0, half)],
                          out_hbm.at[pl.ds(base, half)], so)
    c1.wait()
    s1 = pltpu.async_copy(val_v.at[pl.ds(half, half)],
                          out_hbm.at[pl.ds(base + half, half)], so)
    s0.wait()
    s1.wait()


def kernel(weights, indices):
    return _sc_gather(weights, indices.astype(jnp.int32))
